# baseline (speedup 1.0000x reference)
import functools

import jax
import jax.numpy as jnp
from jax.experimental import pallas as pl
from jax.experimental.pallas import tpu as pltpu

_EPS = 1e-5
_VMEM_LIMIT = 60 * 1024 * 1024


def _fused_kernel(x_ref, w1_ref, b1_ref, g1_ref, bt1_ref,
                  w2_ref, b2_ref, g2_ref, bt2_ref, o_ref,
                  h1_ref, h2_ref, sum1_ref, ssq1_ref, sum2_ref, ssq2_ref,
                  *, n_total, tile, c_blk, t_n, hh_n, w_n):
    p = pl.program_id(0)
    i = pl.program_id(1)
    inv_n = 1.0 / n_total

    @pl.when((p == 0) & (i == 0))
    def _():
        sum1_ref[...] = jnp.zeros_like(sum1_ref)
        ssq1_ref[...] = jnp.zeros_like(ssq1_ref)
        sum2_ref[...] = jnp.zeros_like(sum2_ref)
        ssq2_ref[...] = jnp.zeros_like(ssq2_ref)

    @pl.when(p == 0)
    def _():
        h = jax.lax.dot_general(
            x_ref[...], w1_ref[...], (((1,), (1,)), ((), ())),
            preferred_element_type=jnp.float32) + b1_ref[...]
        h1_ref[pl.ds(i * tile, tile), :] = h.astype(jnp.bfloat16)
        sum1_ref[...] += jnp.sum(h, axis=0, keepdims=True)
        ssq1_ref[...] += jnp.sum(h * h, axis=0, keepdims=True)

    @pl.when(p == 1)
    def _():
        mu = sum1_ref[...] * inv_n
        var = jnp.maximum(ssq1_ref[...] * inv_n - mu * mu, 0.0)
        s = g1_ref[...] * jax.lax.rsqrt(var + _EPS)
        c = bt1_ref[...] - s * mu
        a = jnp.maximum(h1_ref[pl.ds(i * tile, tile), :].astype(jnp.float32)
                        * s + c, 0.0)
        h = jax.lax.dot_general(
            a.astype(jnp.bfloat16), w2_ref[...], (((1,), (1,)), ((), ())),
            preferred_element_type=jnp.float32) + b2_ref[...]
        h2_ref[pl.ds(i * tile, tile), :] = h.astype(jnp.bfloat16)
        sum2_ref[...] += jnp.sum(h, axis=0, keepdims=True)
        ssq2_ref[...] += jnp.sum(h * h, axis=0, keepdims=True)

    @pl.when(p == 2)
    def _():
        mu = sum2_ref[...] * inv_n
        var = jnp.maximum(ssq2_ref[...] * inv_n - mu * mu, 0.0)
        s = g2_ref[...] * jax.lax.rsqrt(var + _EPS)
        c = bt2_ref[...] - s * mu
        res = jnp.maximum(
            h2_ref[pl.ds(i * tile, tile), :].astype(jnp.float32) * s + c, 0.0)
        for t in range(t_n):
            for hh in range(hh_n):
                o_ref[:, t * hh_n + hh, :] = (
                    res[t * c_blk:(t + 1) * c_blk, hh * w_n:(hh + 1) * w_n])


def kernel(x, w1, b1, g1, bt1, w2, b2, g2, bt2):
    B, C, H, W = x.shape
    N = B * H * W
    Fi = w1.shape[0]
    Fo = w2.shape[0]

    t_n = (H * W) // C
    hh_n = H // t_n
    w_n = W
    c_blk = min(256, Fo)
    tile = t_n * c_blk
    cpb = Fo // c_blk
    nsteps = N // tile
    grid = (3, nsteps)
    last = nsteps - 1

    xn = (x.reshape(B, C // c_blk, c_blk, t_n, hh_n * w_n)
          .transpose(0, 1, 3, 2, 4)
          .reshape(N, C).astype(jnp.bfloat16))
    w1b = w1.astype(jnp.bfloat16)
    w2b = w2.astype(jnp.bfloat16)
    b1r = b1.reshape(1, Fi)
    g1r = g1.reshape(1, Fi)
    bt1r = bt1.reshape(1, Fi)
    b2r = b2.reshape(1, Fo)
    g2r = g2.reshape(1, Fo)
    bt2r = bt2.reshape(1, Fo)

    def const_spec(a):
        return pl.BlockSpec(a.shape, lambda p, i: (0, 0))

    x_spec = pl.BlockSpec((tile, C),
                          lambda p, i: (jnp.where(p == 0, i, last), 0))
    o_spec = pl.BlockSpec(
        (None, c_blk, H, W),
        lambda p, i: (jnp.where(p == 2, i // cpb, 0),
                      jnp.where(p == 2, i % cpb, 0), 0, 0))

    out4 = pl.pallas_call(
        functools.partial(_fused_kernel, n_total=N, tile=tile, c_blk=c_blk,
                          t_n=t_n, hh_n=hh_n, w_n=w_n),
        grid=grid,
        in_specs=[x_spec, const_spec(w1b), const_spec(b1r), const_spec(g1r),
                  const_spec(bt1r), const_spec(w2b), const_spec(b2r),
                  const_spec(g2r), const_spec(bt2r)],
        out_specs=o_spec,
        out_shape=jax.ShapeDtypeStruct((B, Fo, H, W), jnp.float32),
        scratch_shapes=[
            pltpu.VMEM((N, Fi), jnp.bfloat16),
            pltpu.VMEM((N, Fo), jnp.bfloat16),
            pltpu.VMEM((1, Fi), jnp.float32),
            pltpu.VMEM((1, Fi), jnp.float32),
            pltpu.VMEM((1, Fo), jnp.float32),
            pltpu.VMEM((1, Fo), jnp.float32),
        ],
        compiler_params=pltpu.CompilerParams(
            dimension_semantics=("arbitrary", "arbitrary"),
            vmem_limit_bytes=_VMEM_LIMIT),
    )(xn, w1b, b1r, g1r, bt1r, w2b, b2r, g2r, bt2r)

    return out4


# speedup vs baseline: 1.2883x; 1.2883x over previous
import functools

import jax
import jax.numpy as jnp
from jax.experimental import pallas as pl
from jax.experimental.pallas import tpu as pltpu

_EPS = 1e-5
_VMEM_LIMIT = 60 * 1024 * 1024


def _pick_tile(n: int) -> int:
    t = min(n, 2048)
    while t > 8 and n % t:
        t //= 2
    return t


def _fused_kernel(x_ref, w1_ref, b1_ref, g1_ref, bt1_ref,
                  w2_ref, b2_ref, g2_ref, bt2_ref, o_ref,
                  h1_ref, h2_ref, w1b_ref, w2b_ref,
                  sum1_ref, ssq1_ref, sum2_ref, ssq2_ref,
                  *, n_total, tile):
    p = pl.program_id(0)
    i = pl.program_id(1)
    inv_n = 1.0 / n_total

    @pl.when((p == 0) & (i == 0))
    def _():
        sum1_ref[...] = jnp.zeros_like(sum1_ref)
        ssq1_ref[...] = jnp.zeros_like(ssq1_ref)
        sum2_ref[...] = jnp.zeros_like(sum2_ref)
        ssq2_ref[...] = jnp.zeros_like(ssq2_ref)
        w1b_ref[...] = w1_ref[...].astype(jnp.bfloat16)
        w2b_ref[...] = w2_ref[...].astype(jnp.bfloat16)

    @pl.when(p == 0)
    def _():
        h = jax.lax.dot_general(
            x_ref[...], w1b_ref[...], (((1,), (1,)), ((), ())),
            preferred_element_type=jnp.float32) + b1_ref[...]
        h1_ref[pl.ds(i * tile, tile), :] = h.astype(jnp.bfloat16)
        sum1_ref[...] += jnp.sum(h, axis=0, keepdims=True)
        ssq1_ref[...] += jnp.sum(h * h, axis=0, keepdims=True)

    @pl.when(p == 1)
    def _():
        mu = sum1_ref[...] * inv_n
        var = jnp.maximum(ssq1_ref[...] * inv_n - mu * mu, 0.0)
        s = g1_ref[...] * jax.lax.rsqrt(var + _EPS)
        c = bt1_ref[...] - s * mu
        a = jnp.maximum(h1_ref[pl.ds(i * tile, tile), :].astype(jnp.float32)
                        * s + c, 0.0)
        h = jax.lax.dot_general(
            a.astype(jnp.bfloat16), w2b_ref[...], (((1,), (1,)), ((), ())),
            preferred_element_type=jnp.float32) + b2_ref[...]
        h2_ref[pl.ds(i * tile, tile), :] = h.astype(jnp.bfloat16)
        sum2_ref[...] += jnp.sum(h, axis=0, keepdims=True)
        ssq2_ref[...] += jnp.sum(h * h, axis=0, keepdims=True)

    @pl.when(p == 2)
    def _():
        mu = sum2_ref[...] * inv_n
        var = jnp.maximum(ssq2_ref[...] * inv_n - mu * mu, 0.0)
        s = g2_ref[...] * jax.lax.rsqrt(var + _EPS)
        c = bt2_ref[...] - s * mu
        o_ref[...] = jnp.maximum(
            h2_ref[pl.ds(i * tile, tile), :].astype(jnp.float32) * s + c, 0.0)


def kernel(x, w1, b1, g1, bt1, w2, b2, g2, bt2):
    B, C, H, W = x.shape
    N = B * H * W
    Fi = w1.shape[0]
    Fo = w2.shape[0]

    xn = jnp.reshape(x, (N, C)).astype(jnp.bfloat16)
    b1r = b1.reshape(1, Fi)
    g1r = g1.reshape(1, Fi)
    bt1r = bt1.reshape(1, Fi)
    b2r = b2.reshape(1, Fo)
    g2r = g2.reshape(1, Fo)
    bt2r = bt2.reshape(1, Fo)

    tile = _pick_tile(N)
    nsteps = N // tile
    grid = (3, nsteps)
    last = nsteps - 1

    def const_spec(a):
        return pl.BlockSpec(a.shape, lambda p, i: (0, 0))

    x_spec = pl.BlockSpec((tile, C),
                          lambda p, i: (jnp.where(p == 0, i, last), 0))
    o_spec = pl.BlockSpec((tile, Fo),
                          lambda p, i: (jnp.where(p == 2, i, 0), 0))

    out_n = pl.pallas_call(
        functools.partial(_fused_kernel, n_total=N, tile=tile),
        grid=grid,
        in_specs=[x_spec, const_spec(w1), const_spec(b1r), const_spec(g1r),
                  const_spec(bt1r), const_spec(w2), const_spec(b2r),
                  const_spec(g2r), const_spec(bt2r)],
        out_specs=o_spec,
        out_shape=jax.ShapeDtypeStruct((N, Fo), jnp.float32),
        scratch_shapes=[
            pltpu.VMEM((N, Fi), jnp.bfloat16),
            pltpu.VMEM((N, Fo), jnp.bfloat16),
            pltpu.VMEM((Fi, C), jnp.bfloat16),
            pltpu.VMEM((Fo, Fi), jnp.bfloat16),
            pltpu.VMEM((1, Fi), jnp.float32),
            pltpu.VMEM((1, Fi), jnp.float32),
            pltpu.VMEM((1, Fo), jnp.float32),
            pltpu.VMEM((1, Fo), jnp.float32),
        ],
        compiler_params=pltpu.CompilerParams(
            dimension_semantics=("arbitrary", "arbitrary"),
            vmem_limit_bytes=_VMEM_LIMIT),
    )(xn, w1, b1r, g1r, bt1r, w2, b2r, g2r, bt2r)

    return jnp.maximum(jnp.reshape(out_n, (B, Fo, H, W)), 0.0)


# speedup vs baseline: 1.3839x; 1.0742x over previous
import functools

import jax
import jax.numpy as jnp
from jax.experimental import pallas as pl
from jax.experimental.pallas import tpu as pltpu

_EPS = 1e-5
_VMEM_LIMIT = 60 * 1024 * 1024


def _pick_tile(n: int) -> int:
    t = min(n, 2048)
    while t > 8 and n % t:
        t //= 2
    return t


def _fused_kernel(x_ref, w1_ref, b1_ref, g1_ref, bt1_ref,
                  w2_ref, b2_ref, g2_ref, bt2_ref, o_ref,
                  h1_ref, h2_ref, w1b_ref, w2b_ref,
                  sum1_ref, ssq1_ref, sum2_ref, ssq2_ref,
                  *, n_total, tile):
    p = pl.program_id(0)
    i = pl.program_id(1)
    inv_n = 1.0 / n_total

    @pl.when((p == 0) & (i == 0))
    def _():
        sum1_ref[...] = jnp.zeros_like(sum1_ref)
        ssq1_ref[...] = jnp.zeros_like(ssq1_ref)
        sum2_ref[...] = jnp.zeros_like(sum2_ref)
        ssq2_ref[...] = jnp.zeros_like(ssq2_ref)
        w1b_ref[...] = w1_ref[...].astype(jnp.bfloat16)
        w2b_ref[...] = w2_ref[...].astype(jnp.bfloat16)

    @pl.when(p == 0)
    def _():
        h = jax.lax.dot_general(
            x_ref[...], w1b_ref[...], (((1,), (1,)), ((), ())),
            preferred_element_type=jnp.float32) + b1_ref[...]
        h1_ref[pl.ds(i * tile, tile), :] = h.astype(jnp.bfloat16)
        sum1_ref[...] += jnp.sum(h, axis=0, keepdims=True)
        ssq1_ref[...] += jnp.sum(h * h, axis=0, keepdims=True)

    @pl.when(p == 1)
    def _():
        mu = sum1_ref[...] * inv_n
        var = jnp.maximum(ssq1_ref[...] * inv_n - mu * mu, 0.0)
        s = g1_ref[...] * jax.lax.rsqrt(var + _EPS)
        c = bt1_ref[...] - s * mu
        a = jnp.maximum(h1_ref[pl.ds(i * tile, tile), :].astype(jnp.float32)
                        * s + c, 0.0)
        h = jax.lax.dot_general(
            a.astype(jnp.bfloat16), w2b_ref[...], (((1,), (1,)), ((), ())),
            preferred_element_type=jnp.float32) + b2_ref[...]
        h2_ref[pl.ds(i * tile, tile), :] = h.astype(jnp.bfloat16)
        sum2_ref[...] += jnp.sum(h, axis=0, keepdims=True)
        ssq2_ref[...] += jnp.sum(h * h, axis=0, keepdims=True)

    @pl.when(p == 2)
    def _():
        mu = sum2_ref[...] * inv_n
        var = jnp.maximum(ssq2_ref[...] * inv_n - mu * mu, 0.0)
        s = g2_ref[...] * jax.lax.rsqrt(var + _EPS)
        c = bt2_ref[...] - s * mu
        o_ref[...] = jnp.maximum(
            h2_ref[pl.ds(i * tile, tile), :].astype(jnp.float32) * s + c, 0.0)


def kernel(x, w1, b1, g1, bt1, w2, b2, g2, bt2):
    B, C, H, W = x.shape
    N = B * H * W
    Fi = w1.shape[0]
    Fo = w2.shape[0]

    xn = jnp.reshape(x, (N, C)).astype(jnp.bfloat16)
    b1r = b1.reshape(1, Fi)
    g1r = g1.reshape(1, Fi)
    bt1r = bt1.reshape(1, Fi)
    b2r = b2.reshape(1, Fo)
    g2r = g2.reshape(1, Fo)
    bt2r = bt2.reshape(1, Fo)

    tile = _pick_tile(N)
    nsteps = N // tile
    grid = (3, nsteps)
    last = nsteps - 1

    def const_spec(a):
        return pl.BlockSpec(a.shape, lambda p, i: (0, 0))

    x_spec = pl.BlockSpec((tile, C),
                          lambda p, i: (jnp.where(p == 0, i, last), 0))
    o_spec = pl.BlockSpec((tile, Fo),
                          lambda p, i: (jnp.where(p == 2, i, 0), 0))

    out_n = pl.pallas_call(
        functools.partial(_fused_kernel, n_total=N, tile=tile),
        grid=grid,
        in_specs=[x_spec, const_spec(w1), const_spec(b1r), const_spec(g1r),
                  const_spec(bt1r), const_spec(w2), const_spec(b2r),
                  const_spec(g2r), const_spec(bt2r)],
        out_specs=o_spec,
        out_shape=jax.ShapeDtypeStruct((N, Fo), jnp.float32),
        scratch_shapes=[
            pltpu.VMEM((N, Fi), jnp.bfloat16),
            pltpu.VMEM((N, Fo), jnp.bfloat16),
            pltpu.VMEM((Fi, C), jnp.bfloat16),
            pltpu.VMEM((Fo, Fi), jnp.bfloat16),
            pltpu.VMEM((1, Fi), jnp.float32),
            pltpu.VMEM((1, Fi), jnp.float32),
            pltpu.VMEM((1, Fo), jnp.float32),
            pltpu.VMEM((1, Fo), jnp.float32),
        ],
        compiler_params=pltpu.CompilerParams(
            dimension_semantics=("arbitrary", "arbitrary"),
            vmem_limit_bytes=_VMEM_LIMIT),
    )(xn, w1, b1r, g1r, bt1r, w2, b2r, g2r, bt2r)

    return jnp.reshape(out_n, (B, Fo, H, W))
